# baseline (speedup 1.0000x reference)
import functools

import jax
import jax.numpy as jnp
from jax import lax
from jax.experimental import pallas as pl
from jax.experimental.pallas import tpu as pltpu, tpu_sc as plsc

N_NODES = 10000
N_EDGES = 160000
D_NODE = 256
D_EDGE = 64
HIDDEN = 256
N_EXPERTS = 3

SC_CORES = 2
SC_SUBCORES = 16
SC_WORKERS = SC_CORES * SC_SUBCORES

CHUNK = 128
NSLAB = 4
CHUNKS_PER_TILE = 10
SLAB = SC_WORKERS * CHUNKS_PER_TILE * CHUNK
E_PAD = NSLAB * SLAB
EDGES_PER_TILE = CHUNKS_PER_TILE * CHUNK
BE = 640
NODE_BLK = 200


def _gelu_exact(x):
    return 0.5 * x * (1.0 + lax.erf(x * 0.7071067811865476))


HALF = HIDDEN // 2


def _pack_bf16_pair(lo_f32, hi_f32):
    def rne_bits(y):
        b = lax.bitcast_convert_type(y, jnp.int32)
        return b + jnp.int32(0x7FFF) + ((b >> 16) & jnp.int32(1))

    lo16 = (rne_bits(lo_f32) >> 16) & jnp.int32(0xFFFF)
    hi16 = rne_bits(hi_f32) & jnp.int32(-65536)
    return hi16 | lo16


def _unpack_bf16_pair(w):
    lo = lax.bitcast_convert_type(w << 16, jnp.float32)
    hi = lax.bitcast_convert_type(w & jnp.int32(-65536), jnp.float32)
    return lo, hi


def _proj_body(n_ref, w_ref, o_ref):
    y = jnp.dot(n_ref[...], w_ref[0], preferred_element_type=jnp.float32)
    o_ref[...] = _pack_bf16_pair(y[:, :HALF], y[:, HALF:])[None]


def _project_nodes(node_embed, w_stacked):
    return pl.pallas_call(
        _proj_body,
        grid=(2 * N_EXPERTS, N_NODES // NODE_BLK),
        in_specs=[
            pl.BlockSpec((NODE_BLK, D_NODE), lambda j, i: (i, 0)),
            pl.BlockSpec((1, D_NODE, HIDDEN), lambda j, i: (j, 0, 0)),
        ],
        out_specs=pl.BlockSpec((1, NODE_BLK, HALF), lambda j, i: (j, i, 0)),
        out_shape=jax.ShapeDtypeStruct((2 * N_EXPERTS, N_NODES, HALF),
                                       jnp.int32),
    )(node_embed, w_stacked)


def _sc_gather_body(slab_base, table_hbm, src_hbm, dst_hbm, type_hbm,
                    hs_hbm, hd_hbm,
                    srcb, dstb, typeb, idxs, idxd,
                    rows_s0, rows_d0, rows_s1, rows_d1,
                    sem_s0, sem_d0, sem_s1, sem_d1):
    wid = lax.axis_index("s") * SC_CORES + lax.axis_index("c")
    tile_base = slab_base + wid * EDGES_PER_TILE
    out_base = wid * EDGES_PER_TILE

    pltpu.sync_copy(src_hbm.at[pl.ds(tile_base, EDGES_PER_TILE)], srcb)
    pltpu.sync_copy(dst_hbm.at[pl.ds(tile_base, EDGES_PER_TILE)], dstb)
    pltpu.sync_copy(type_hbm.at[pl.ds(tile_base, EDGES_PER_TILE)], typeb)

    def idx_body(i, carry):
        sl = pl.ds(i * 16, 16)
        t16 = typeb[sl] * N_NODES
        idxs[sl] = t16 + srcb[sl]
        idxd[sl] = t16 + dstb[sl] + N_EXPERTS * N_NODES
        return carry

    lax.fori_loop(0, EDGES_PER_TILE // 16, idx_body, 0)

    zeros16 = jnp.zeros((16,), jnp.int32)
    for i in range(CHUNK // 16):
        idxs[pl.ds(EDGES_PER_TILE + i * 16, 16)] = zeros16
        idxd[pl.ds(EDGES_PER_TILE + i * 16, 16)] = zeros16

    def gather(c, rs, rd, ss, sd):
        csl = pl.ds(c * CHUNK, CHUNK)
        cs = pltpu.async_copy(table_hbm.at[idxs.at[csl]], rs, ss)
        cd = pltpu.async_copy(table_hbm.at[idxd.at[csl]], rd, sd)
        return cs, cd

    gather(0, rows_s0, rows_d0, sem_s0, sem_d0)

    def pair_body(p, carry):
        c0 = 2 * p
        gather(c0 + 1, rows_s1, rows_d1, sem_s1, sem_d1)
        pltpu.make_async_copy(table_hbm.at[pl.ds(0, CHUNK)], rows_s0, sem_s0).wait()
        pltpu.make_async_copy(table_hbm.at[pl.ds(0, CHUNK)], rows_d0, sem_d0).wait()
        base0 = out_base + c0 * CHUNK
        pltpu.sync_copy(rows_s0, hs_hbm.at[pl.ds(base0, CHUNK)])
        pltpu.sync_copy(rows_d0, hd_hbm.at[pl.ds(base0, CHUNK)])
        gather(c0 + 2, rows_s0, rows_d0, sem_s0, sem_d0)
        pltpu.make_async_copy(table_hbm.at[pl.ds(0, CHUNK)], rows_s1, sem_s1).wait()
        pltpu.make_async_copy(table_hbm.at[pl.ds(0, CHUNK)], rows_d1, sem_d1).wait()
        base1 = base0 + CHUNK
        pltpu.sync_copy(rows_s1, hs_hbm.at[pl.ds(base1, CHUNK)])
        pltpu.sync_copy(rows_d1, hd_hbm.at[pl.ds(base1, CHUNK)])
        return carry

    lax.fori_loop(0, CHUNKS_PER_TILE // 2, pair_body, 0)

    pltpu.make_async_copy(table_hbm.at[pl.ds(0, CHUNK)], rows_s0, sem_s0).wait()
    pltpu.make_async_copy(table_hbm.at[pl.ds(0, CHUNK)], rows_d0, sem_d0).wait()


def _sc_gather(table, src, dst, etype, slab_base):
    mesh = plsc.VectorSubcoreMesh(core_axis_name="c", subcore_axis_name="s")
    kern = pl.kernel(
        functools.partial(_sc_gather_body, slab_base),
        out_type=(
            jax.ShapeDtypeStruct((SLAB, HALF), jnp.int32),
            jax.ShapeDtypeStruct((SLAB, HALF), jnp.int32),
        ),
        mesh=mesh,
        scratch_types=[
            pltpu.VMEM((EDGES_PER_TILE,), jnp.int32),
            pltpu.VMEM((EDGES_PER_TILE,), jnp.int32),
            pltpu.VMEM((EDGES_PER_TILE,), jnp.int32),
            pltpu.VMEM((EDGES_PER_TILE + CHUNK,), jnp.int32),
            pltpu.VMEM((EDGES_PER_TILE + CHUNK,), jnp.int32),
            pltpu.VMEM((CHUNK, HALF), jnp.int32),
            pltpu.VMEM((CHUNK, HALF), jnp.int32),
            pltpu.VMEM((CHUNK, HALF), jnp.int32),
            pltpu.VMEM((CHUNK, HALF), jnp.int32),
            pltpu.SemaphoreType.DMA,
            pltpu.SemaphoreType.DMA,
            pltpu.SemaphoreType.DMA,
            pltpu.SemaphoreType.DMA,
        ],
    )
    return kern(table, src, dst, etype)


def _edge_body(hs_ref, hd_ref, es_ref, t_ref, bz_ref,
               w1c_ref, b1_ref, w2_ref, b2_ref, w3_ref, b3_ref,
               dz_ref, rho_ref):
    t = t_ref[...]
    hs_lo, hs_hi = _unpack_bf16_pair(hs_ref[...])
    hd_lo, hd_hi = _unpack_bf16_pair(hd_ref[...])
    hsum_lo = hs_lo + hd_lo
    hsum_hi = hs_hi + hd_hi

    def sel(parts):
        acc = jnp.where(t == 0.0, parts[0], 0.0)
        for e in range(1, N_EXPERTS):
            acc = acc + jnp.where(t == float(e), parts[e], 0.0)
        return acc

    pe = jnp.dot(es_ref[...], w1c_ref[...], preferred_element_type=jnp.float32)
    pe1 = sel([pe[:, e * HIDDEN:(e + 1) * HIDDEN] + b1_ref[e][None, :]
               for e in range(N_EXPERTS)])
    h1_lo = _gelu_exact(hsum_lo + pe1[:, :HALF])
    h1_hi = _gelu_exact(hsum_hi + pe1[:, HALF:])

    y = (jnp.dot(h1_lo, w2_ref[:HALF], preferred_element_type=jnp.float32) +
         jnp.dot(h1_hi, w2_ref[HALF:], preferred_element_type=jnp.float32))
    y1 = sel([y[:, e * HIDDEN:(e + 1) * HIDDEN] + b2_ref[e][None, :]
              for e in range(N_EXPERTS)])
    g = _gelu_exact(y1)

    z3 = jnp.dot(g, w3_ref[...], preferred_element_type=jnp.float32)
    delta = sel([z3[:, e:e + 1] + b3_ref[e, 0] for e in range(N_EXPERTS)])

    dz_ref[...] = delta
    rho_ref[...] = jnp.tanh(bz_ref[...] + delta)


def _edge_mlp(n_edges, blk0, hs, hd, edge_state, t3, bz3,
              w1c, b1, w2, b2, w3c, b3):
    full = lambda s: pl.BlockSpec(s, lambda i: tuple(0 for _ in s))
    return pl.pallas_call(
        _edge_body,
        grid=(n_edges // BE,),
        in_specs=[
            pl.BlockSpec((BE, HALF), lambda i: (i, 0)),
            pl.BlockSpec((BE, HALF), lambda i: (i, 0)),
            pl.BlockSpec((BE, D_EDGE), lambda i: (i + blk0, 0)),
            pl.BlockSpec((BE, 1), lambda i: (i + blk0, 0)),
            pl.BlockSpec((BE, 1), lambda i: (i + blk0, 0)),
            full((D_EDGE, N_EXPERTS * HIDDEN)),
            full((N_EXPERTS, HIDDEN)),
            full((HIDDEN, N_EXPERTS * HIDDEN)),
            full((N_EXPERTS, HIDDEN)),
            full((HIDDEN, N_EXPERTS)),
            full((N_EXPERTS, 1)),
        ],
        out_specs=[
            pl.BlockSpec((BE, 1), lambda i: (i, 0)),
            pl.BlockSpec((BE, 1), lambda i: (i, 0)),
        ],
        out_shape=[
            jax.ShapeDtypeStruct((n_edges, 1), jnp.float32),
            jax.ShapeDtypeStruct((n_edges, 1), jnp.float32),
        ],
    )(hs, hd, edge_state, t3, bz3, w1c, b1, w2, b2, w3c, b3)


def kernel(node_embed, edge_state, edge_index, edge_type, baseline_z,
           W1, b1, W2, b2, W3, b3):
    pad = (0, E_PAD - N_EDGES)
    src = jnp.pad(edge_index[0].astype(jnp.int32), pad)
    dst = jnp.pad(edge_index[1].astype(jnp.int32), pad)
    etype = edge_type.astype(jnp.int32)
    etype_p = jnp.pad(etype, pad)

    w_stacked = jnp.concatenate([W1[:, :D_NODE, :], W1[:, D_NODE:2 * D_NODE, :]],
                                axis=0)
    table = _project_nodes(node_embed, w_stacked).reshape(
        2 * N_EXPERTS * N_NODES, HALF)

    w1c = W1[:, 2 * D_NODE:, :].transpose(1, 0, 2).reshape(
        D_EDGE, N_EXPERTS * HIDDEN)
    w2c = W2.transpose(1, 0, 2).reshape(HIDDEN, N_EXPERTS * HIDDEN)
    w3c = W3[:, :, 0].T
    t3 = etype.astype(jnp.float32).reshape(N_EDGES, 1)
    bz3 = baseline_z.reshape(N_EDGES, 1)

    dz_parts, rho_parts = [], []
    for s in range(NSLAB):
        lo = s * SLAB
        n = min(SLAB, N_EDGES - lo)
        hs, hd = _sc_gather(table, src, dst, etype_p, lo)
        dz_s, rho_s = _edge_mlp(n, lo // BE, hs, hd, edge_state, t3, bz3,
                                w1c, b1, w2c, b2, w3c, b3)
        dz_parts.append(dz_s)
        rho_parts.append(rho_s)

    dz3 = jnp.concatenate(dz_parts, axis=0)
    rho3 = jnp.concatenate(rho_parts, axis=0)
    return dz3.reshape(N_EDGES), rho3.reshape(N_EDGES)


# speedup vs baseline: 1.5927x; 1.5927x over previous
import functools

import jax
import jax.numpy as jnp
from jax import lax
from jax.experimental import pallas as pl
from jax.experimental.pallas import tpu as pltpu, tpu_sc as plsc

N_NODES = 10000
N_EDGES = 160000
D_NODE = 256
D_EDGE = 64
HIDDEN = 256
N_EXPERTS = 3

SC_CORES = 2
SC_SUBCORES = 16
SC_WORKERS = SC_CORES * SC_SUBCORES

CHUNK = 128
NSLAB = 1
CHUNKS_PER_TILE = 40
SLAB = SC_WORKERS * CHUNKS_PER_TILE * CHUNK
E_PAD = NSLAB * SLAB
EDGES_PER_TILE = CHUNKS_PER_TILE * CHUNK
BE = 640
NODE_BLK = 200


def _gelu_exact(x):
    return 0.5 * x * (1.0 + lax.erf(x * 0.7071067811865476))


HALF = HIDDEN // 2


def _pack_bf16_pair(lo_f32, hi_f32):
    def rne_bits(y):
        b = lax.bitcast_convert_type(y, jnp.int32)
        return b + jnp.int32(0x7FFF) + ((b >> 16) & jnp.int32(1))

    lo16 = (rne_bits(lo_f32) >> 16) & jnp.int32(0xFFFF)
    hi16 = rne_bits(hi_f32) & jnp.int32(-65536)
    return hi16 | lo16


def _unpack_bf16_pair(w):
    lo = lax.bitcast_convert_type(w << 16, jnp.float32)
    hi = lax.bitcast_convert_type(w & jnp.int32(-65536), jnp.float32)
    return lo, hi


def _proj_body(n_ref, w_ref, o_ref):
    y = jnp.dot(n_ref[...], w_ref[0], preferred_element_type=jnp.float32)
    o_ref[...] = _pack_bf16_pair(y[:, :HALF], y[:, HALF:])[None]


def _project_nodes(node_embed, w_stacked):
    return pl.pallas_call(
        _proj_body,
        grid=(2 * N_EXPERTS, N_NODES // NODE_BLK),
        in_specs=[
            pl.BlockSpec((NODE_BLK, D_NODE), lambda j, i: (i, 0)),
            pl.BlockSpec((1, D_NODE, HIDDEN), lambda j, i: (j, 0, 0)),
        ],
        out_specs=pl.BlockSpec((1, NODE_BLK, HALF), lambda j, i: (j, i, 0)),
        out_shape=jax.ShapeDtypeStruct((2 * N_EXPERTS, N_NODES, HALF),
                                       jnp.int32),
    )(node_embed, w_stacked)


def _sc_gather_body(slab_base, table_hbm, src_hbm, dst_hbm, type_hbm,
                    hs_hbm, hd_hbm,
                    srcb, dstb, typeb, idxs, idxd,
                    rows_s0, rows_d0, rows_s1, rows_d1,
                    sem_s0, sem_d0, sem_s1, sem_d1):
    wid = lax.axis_index("s") * SC_CORES + lax.axis_index("c")
    tile_base = slab_base + wid * EDGES_PER_TILE
    out_base = wid * EDGES_PER_TILE

    pltpu.sync_copy(src_hbm.at[pl.ds(tile_base, EDGES_PER_TILE)], srcb)
    pltpu.sync_copy(dst_hbm.at[pl.ds(tile_base, EDGES_PER_TILE)], dstb)
    pltpu.sync_copy(type_hbm.at[pl.ds(tile_base, EDGES_PER_TILE)], typeb)

    def idx_body(i, carry):
        sl = pl.ds(i * 16, 16)
        t16 = typeb[sl] * N_NODES
        idxs[sl] = t16 + srcb[sl]
        idxd[sl] = t16 + dstb[sl] + N_EXPERTS * N_NODES
        return carry

    lax.fori_loop(0, EDGES_PER_TILE // 16, idx_body, 0)

    zeros16 = jnp.zeros((16,), jnp.int32)
    for i in range(CHUNK // 16):
        idxs[pl.ds(EDGES_PER_TILE + i * 16, 16)] = zeros16
        idxd[pl.ds(EDGES_PER_TILE + i * 16, 16)] = zeros16

    def gather(c, rs, rd, ss, sd):
        csl = pl.ds(c * CHUNK, CHUNK)
        cs = pltpu.async_copy(table_hbm.at[idxs.at[csl]], rs, ss)
        cd = pltpu.async_copy(table_hbm.at[idxd.at[csl]], rd, sd)
        return cs, cd

    gather(0, rows_s0, rows_d0, sem_s0, sem_d0)

    def pair_body(p, carry):
        c0 = 2 * p
        gather(c0 + 1, rows_s1, rows_d1, sem_s1, sem_d1)
        pltpu.make_async_copy(table_hbm.at[pl.ds(0, CHUNK)], rows_s0, sem_s0).wait()
        pltpu.make_async_copy(table_hbm.at[pl.ds(0, CHUNK)], rows_d0, sem_d0).wait()
        base0 = out_base + c0 * CHUNK
        pltpu.sync_copy(rows_s0, hs_hbm.at[pl.ds(base0, CHUNK)])
        pltpu.sync_copy(rows_d0, hd_hbm.at[pl.ds(base0, CHUNK)])
        gather(c0 + 2, rows_s0, rows_d0, sem_s0, sem_d0)
        pltpu.make_async_copy(table_hbm.at[pl.ds(0, CHUNK)], rows_s1, sem_s1).wait()
        pltpu.make_async_copy(table_hbm.at[pl.ds(0, CHUNK)], rows_d1, sem_d1).wait()
        base1 = base0 + CHUNK
        pltpu.sync_copy(rows_s1, hs_hbm.at[pl.ds(base1, CHUNK)])
        pltpu.sync_copy(rows_d1, hd_hbm.at[pl.ds(base1, CHUNK)])
        return carry

    lax.fori_loop(0, CHUNKS_PER_TILE // 2, pair_body, 0)

    pltpu.make_async_copy(table_hbm.at[pl.ds(0, CHUNK)], rows_s0, sem_s0).wait()
    pltpu.make_async_copy(table_hbm.at[pl.ds(0, CHUNK)], rows_d0, sem_d0).wait()


def _sc_gather(table, src, dst, etype, slab_base):
    mesh = plsc.VectorSubcoreMesh(core_axis_name="c", subcore_axis_name="s")
    kern = pl.kernel(
        functools.partial(_sc_gather_body, slab_base),
        out_type=(
            jax.ShapeDtypeStruct((SLAB, HALF), jnp.int32),
            jax.ShapeDtypeStruct((SLAB, HALF), jnp.int32),
        ),
        mesh=mesh,
        scratch_types=[
            pltpu.VMEM((EDGES_PER_TILE,), jnp.int32),
            pltpu.VMEM((EDGES_PER_TILE,), jnp.int32),
            pltpu.VMEM((EDGES_PER_TILE,), jnp.int32),
            pltpu.VMEM((EDGES_PER_TILE + CHUNK,), jnp.int32),
            pltpu.VMEM((EDGES_PER_TILE + CHUNK,), jnp.int32),
            pltpu.VMEM((CHUNK, HALF), jnp.int32),
            pltpu.VMEM((CHUNK, HALF), jnp.int32),
            pltpu.VMEM((CHUNK, HALF), jnp.int32),
            pltpu.VMEM((CHUNK, HALF), jnp.int32),
            pltpu.SemaphoreType.DMA,
            pltpu.SemaphoreType.DMA,
            pltpu.SemaphoreType.DMA,
            pltpu.SemaphoreType.DMA,
        ],
    )
    return kern(table, src, dst, etype)


def _edge_body(hs_ref, hd_ref, es_ref, t_ref, bz_ref,
               w1c_ref, b1_ref, w2_ref, b2_ref, w3_ref, b3_ref,
               dz_ref, rho_ref):
    t = t_ref[...]
    hs_lo, hs_hi = _unpack_bf16_pair(hs_ref[...])
    hd_lo, hd_hi = _unpack_bf16_pair(hd_ref[...])
    hsum_lo = hs_lo + hd_lo
    hsum_hi = hs_hi + hd_hi

    def sel(parts):
        acc = jnp.where(t == 0.0, parts[0], 0.0)
        for e in range(1, N_EXPERTS):
            acc = acc + jnp.where(t == float(e), parts[e], 0.0)
        return acc

    pe = jnp.dot(es_ref[...], w1c_ref[...], preferred_element_type=jnp.float32)
    pe1 = sel([pe[:, e * HIDDEN:(e + 1) * HIDDEN] + b1_ref[e][None, :]
               for e in range(N_EXPERTS)])
    h1_lo = _gelu_exact(hsum_lo + pe1[:, :HALF]).astype(jnp.bfloat16)
    h1_hi = _gelu_exact(hsum_hi + pe1[:, HALF:]).astype(jnp.bfloat16)

    y = (jnp.dot(h1_lo, w2_ref[:HALF], preferred_element_type=jnp.float32) +
         jnp.dot(h1_hi, w2_ref[HALF:], preferred_element_type=jnp.float32))
    y1 = sel([y[:, e * HIDDEN:(e + 1) * HIDDEN] + b2_ref[e][None, :]
              for e in range(N_EXPERTS)])
    g = _gelu_exact(y1)

    z3 = jnp.dot(g, w3_ref[...], preferred_element_type=jnp.float32)
    delta = sel([z3[:, e:e + 1] + b3_ref[e, 0] for e in range(N_EXPERTS)])

    dz_ref[...] = delta
    rho_ref[...] = jnp.tanh(bz_ref[...] + delta)


def _edge_mlp(n_edges, blk0, hs, hd, edge_state, t3, bz3,
              w1c, b1, w2, b2, w3c, b3):
    full = lambda s: pl.BlockSpec(s, lambda i: tuple(0 for _ in s))
    return pl.pallas_call(
        _edge_body,
        grid=(n_edges // BE,),
        in_specs=[
            pl.BlockSpec((BE, HALF), lambda i: (i, 0)),
            pl.BlockSpec((BE, HALF), lambda i: (i, 0)),
            pl.BlockSpec((BE, D_EDGE), lambda i: (i + blk0, 0)),
            pl.BlockSpec((BE, 1), lambda i: (i + blk0, 0)),
            pl.BlockSpec((BE, 1), lambda i: (i + blk0, 0)),
            full((D_EDGE, N_EXPERTS * HIDDEN)),
            full((N_EXPERTS, HIDDEN)),
            full((HIDDEN, N_EXPERTS * HIDDEN)),
            full((N_EXPERTS, HIDDEN)),
            full((HIDDEN, N_EXPERTS)),
            full((N_EXPERTS, 1)),
        ],
        out_specs=[
            pl.BlockSpec((BE, 1), lambda i: (i, 0)),
            pl.BlockSpec((BE, 1), lambda i: (i, 0)),
        ],
        out_shape=[
            jax.ShapeDtypeStruct((n_edges, 1), jnp.float32),
            jax.ShapeDtypeStruct((n_edges, 1), jnp.float32),
        ],
    )(hs, hd, edge_state, t3, bz3, w1c, b1, w2, b2, w3c, b3)


def kernel(node_embed, edge_state, edge_index, edge_type, baseline_z,
           W1, b1, W2, b2, W3, b3):
    pad = (0, E_PAD - N_EDGES)
    src = jnp.pad(edge_index[0].astype(jnp.int32), pad)
    dst = jnp.pad(edge_index[1].astype(jnp.int32), pad)
    etype = edge_type.astype(jnp.int32)
    etype_p = jnp.pad(etype, pad)

    w_stacked = jnp.concatenate([W1[:, :D_NODE, :], W1[:, D_NODE:2 * D_NODE, :]],
                                axis=0)
    table = _project_nodes(node_embed, w_stacked).reshape(
        2 * N_EXPERTS * N_NODES, HALF)

    w1c = W1[:, 2 * D_NODE:, :].transpose(1, 0, 2).reshape(
        D_EDGE, N_EXPERTS * HIDDEN).astype(jnp.bfloat16)
    w2c = W2.transpose(1, 0, 2).reshape(
        HIDDEN, N_EXPERTS * HIDDEN).astype(jnp.bfloat16)
    w3c = W3[:, :, 0].T
    t3 = etype.astype(jnp.float32).reshape(N_EDGES, 1)
    bz3 = baseline_z.reshape(N_EDGES, 1)

    dz_parts, rho_parts = [], []
    for s in range(NSLAB):
        lo = s * SLAB
        n = min(SLAB, N_EDGES - lo)
        hs, hd = _sc_gather(table, src, dst, etype_p, lo)
        dz_s, rho_s = _edge_mlp(n, lo // BE, hs, hd,
                                edge_state.astype(jnp.bfloat16), t3, bz3,
                                w1c, b1, w2c, b2, w3c, b3)
        dz_parts.append(dz_s)
        rho_parts.append(rho_s)

    dz3 = jnp.concatenate(dz_parts, axis=0)
    rho3 = jnp.concatenate(rho_parts, axis=0)
    return dz3.reshape(N_EDGES), rho3.reshape(N_EDGES)


# speedup vs baseline: 1.6294x; 1.0231x over previous
import functools

import jax
import jax.numpy as jnp
from jax import lax
from jax.experimental import pallas as pl
from jax.experimental.pallas import tpu as pltpu, tpu_sc as plsc

N_NODES = 10000
N_EDGES = 160000
D_NODE = 256
D_EDGE = 64
HIDDEN = 256
N_EXPERTS = 3

SC_CORES = 2
SC_SUBCORES = 16
SC_WORKERS = SC_CORES * SC_SUBCORES

CHUNK = 128
NSLAB = 1
CHUNKS_PER_TILE = 40
SLAB = SC_WORKERS * CHUNKS_PER_TILE * CHUNK
E_PAD = NSLAB * SLAB
EDGES_PER_TILE = CHUNKS_PER_TILE * CHUNK
BE = 1280
NODE_BLK = 200


def _gelu_exact(x):
    return 0.5 * x * (1.0 + lax.erf(x * 0.7071067811865476))


HALF = HIDDEN // 2


def _pack_bf16_pair(lo_f32, hi_f32):
    def rne_bits(y):
        b = lax.bitcast_convert_type(y, jnp.int32)
        return b + jnp.int32(0x7FFF) + ((b >> 16) & jnp.int32(1))

    lo16 = (rne_bits(lo_f32) >> 16) & jnp.int32(0xFFFF)
    hi16 = rne_bits(hi_f32) & jnp.int32(-65536)
    return hi16 | lo16


def _unpack_bf16_pair(w):
    lo = lax.bitcast_convert_type(w << 16, jnp.float32)
    hi = lax.bitcast_convert_type(w & jnp.int32(-65536), jnp.float32)
    return lo, hi


def _proj_body(n_ref, w_ref, o_ref):
    y = jnp.dot(n_ref[...], w_ref[0], preferred_element_type=jnp.float32)
    o_ref[...] = _pack_bf16_pair(y[:, :HALF], y[:, HALF:])[None]


def _project_nodes(node_embed, w_stacked):
    return pl.pallas_call(
        _proj_body,
        grid=(2 * N_EXPERTS, N_NODES // NODE_BLK),
        in_specs=[
            pl.BlockSpec((NODE_BLK, D_NODE), lambda j, i: (i, 0)),
            pl.BlockSpec((1, D_NODE, HIDDEN), lambda j, i: (j, 0, 0)),
        ],
        out_specs=pl.BlockSpec((1, NODE_BLK, HALF), lambda j, i: (j, i, 0)),
        out_shape=jax.ShapeDtypeStruct((2 * N_EXPERTS, N_NODES, HALF),
                                       jnp.int32),
    )(node_embed, w_stacked)


def _sc_gather_body(slab_base, table_hbm, src_hbm, dst_hbm, type_hbm,
                    hs_hbm, hd_hbm,
                    srcb, dstb, typeb, idxs, idxd,
                    rows_s0, rows_d0, rows_s1, rows_d1,
                    sem_s0, sem_d0, sem_s1, sem_d1):
    wid = lax.axis_index("s") * SC_CORES + lax.axis_index("c")
    tile_base = slab_base + wid * EDGES_PER_TILE
    out_base = wid * EDGES_PER_TILE

    pltpu.sync_copy(src_hbm.at[pl.ds(tile_base, EDGES_PER_TILE)], srcb)
    pltpu.sync_copy(dst_hbm.at[pl.ds(tile_base, EDGES_PER_TILE)], dstb)
    pltpu.sync_copy(type_hbm.at[pl.ds(tile_base, EDGES_PER_TILE)], typeb)

    def idx_body(i, carry):
        sl = pl.ds(i * 16, 16)
        t16 = typeb[sl] * N_NODES
        idxs[sl] = t16 + srcb[sl]
        idxd[sl] = t16 + dstb[sl] + N_EXPERTS * N_NODES
        return carry

    lax.fori_loop(0, EDGES_PER_TILE // 16, idx_body, 0)

    zeros16 = jnp.zeros((16,), jnp.int32)
    for i in range(CHUNK // 16):
        idxs[pl.ds(EDGES_PER_TILE + i * 16, 16)] = zeros16
        idxd[pl.ds(EDGES_PER_TILE + i * 16, 16)] = zeros16

    def gather(c, rs, rd, ss, sd):
        csl = pl.ds(c * CHUNK, CHUNK)
        cs = pltpu.async_copy(table_hbm.at[idxs.at[csl]], rs, ss)
        cd = pltpu.async_copy(table_hbm.at[idxd.at[csl]], rd, sd)
        return cs, cd

    gather(0, rows_s0, rows_d0, sem_s0, sem_d0)

    def pair_body(p, carry):
        c0 = 2 * p
        gather(c0 + 1, rows_s1, rows_d1, sem_s1, sem_d1)
        pltpu.make_async_copy(table_hbm.at[pl.ds(0, CHUNK)], rows_s0, sem_s0).wait()
        pltpu.make_async_copy(table_hbm.at[pl.ds(0, CHUNK)], rows_d0, sem_d0).wait()
        base0 = out_base + c0 * CHUNK
        pltpu.sync_copy(rows_s0, hs_hbm.at[pl.ds(base0, CHUNK)])
        pltpu.sync_copy(rows_d0, hd_hbm.at[pl.ds(base0, CHUNK)])
        gather(c0 + 2, rows_s0, rows_d0, sem_s0, sem_d0)
        pltpu.make_async_copy(table_hbm.at[pl.ds(0, CHUNK)], rows_s1, sem_s1).wait()
        pltpu.make_async_copy(table_hbm.at[pl.ds(0, CHUNK)], rows_d1, sem_d1).wait()
        base1 = base0 + CHUNK
        pltpu.sync_copy(rows_s1, hs_hbm.at[pl.ds(base1, CHUNK)])
        pltpu.sync_copy(rows_d1, hd_hbm.at[pl.ds(base1, CHUNK)])
        return carry

    lax.fori_loop(0, CHUNKS_PER_TILE // 2, pair_body, 0)

    pltpu.make_async_copy(table_hbm.at[pl.ds(0, CHUNK)], rows_s0, sem_s0).wait()
    pltpu.make_async_copy(table_hbm.at[pl.ds(0, CHUNK)], rows_d0, sem_d0).wait()


def _sc_gather(table, src, dst, etype, slab_base):
    mesh = plsc.VectorSubcoreMesh(core_axis_name="c", subcore_axis_name="s")
    kern = pl.kernel(
        functools.partial(_sc_gather_body, slab_base),
        out_type=(
            jax.ShapeDtypeStruct((SLAB, HALF), jnp.int32),
            jax.ShapeDtypeStruct((SLAB, HALF), jnp.int32),
        ),
        mesh=mesh,
        scratch_types=[
            pltpu.VMEM((EDGES_PER_TILE,), jnp.int32),
            pltpu.VMEM((EDGES_PER_TILE,), jnp.int32),
            pltpu.VMEM((EDGES_PER_TILE,), jnp.int32),
            pltpu.VMEM((EDGES_PER_TILE + CHUNK,), jnp.int32),
            pltpu.VMEM((EDGES_PER_TILE + CHUNK,), jnp.int32),
            pltpu.VMEM((CHUNK, HALF), jnp.int32),
            pltpu.VMEM((CHUNK, HALF), jnp.int32),
            pltpu.VMEM((CHUNK, HALF), jnp.int32),
            pltpu.VMEM((CHUNK, HALF), jnp.int32),
            pltpu.SemaphoreType.DMA,
            pltpu.SemaphoreType.DMA,
            pltpu.SemaphoreType.DMA,
            pltpu.SemaphoreType.DMA,
        ],
    )
    return kern(table, src, dst, etype)


def _edge_body(hs_ref, hd_ref, es_ref, t_ref, bz_ref,
               w1c_ref, b1_ref, w2_ref, b2_ref, w3_ref, b3_ref,
               dz_ref, rho_ref):
    t = t_ref[...]
    hs_lo, hs_hi = _unpack_bf16_pair(hs_ref[...])
    hd_lo, hd_hi = _unpack_bf16_pair(hd_ref[...])
    hsum_lo = hs_lo + hd_lo
    hsum_hi = hs_hi + hd_hi

    def sel(parts):
        acc = jnp.where(t == 0.0, parts[0], 0.0)
        for e in range(1, N_EXPERTS):
            acc = acc + jnp.where(t == float(e), parts[e], 0.0)
        return acc

    pe = jnp.dot(es_ref[...], w1c_ref[...], preferred_element_type=jnp.float32)
    pe1 = sel([pe[:, e * HIDDEN:(e + 1) * HIDDEN] + b1_ref[e][None, :]
               for e in range(N_EXPERTS)])
    h1_lo = _gelu_exact(hsum_lo + pe1[:, :HALF]).astype(jnp.bfloat16)
    h1_hi = _gelu_exact(hsum_hi + pe1[:, HALF:]).astype(jnp.bfloat16)

    y = (jnp.dot(h1_lo, w2_ref[:HALF], preferred_element_type=jnp.float32) +
         jnp.dot(h1_hi, w2_ref[HALF:], preferred_element_type=jnp.float32))
    y1 = sel([y[:, e * HIDDEN:(e + 1) * HIDDEN] + b2_ref[e][None, :]
              for e in range(N_EXPERTS)])
    g = _gelu_exact(y1)

    z3 = jnp.dot(g, w3_ref[...], preferred_element_type=jnp.float32)
    delta = sel([z3[:, e:e + 1] + b3_ref[e, 0] for e in range(N_EXPERTS)])

    dz_ref[...] = delta
    rho_ref[...] = jnp.tanh(bz_ref[...] + delta)


def _edge_mlp(n_edges, blk0, hs, hd, edge_state, t3, bz3,
              w1c, b1, w2, b2, w3c, b3):
    full = lambda s: pl.BlockSpec(s, lambda i: tuple(0 for _ in s))
    return pl.pallas_call(
        _edge_body,
        grid=(n_edges // BE,),
        in_specs=[
            pl.BlockSpec((BE, HALF), lambda i: (i, 0)),
            pl.BlockSpec((BE, HALF), lambda i: (i, 0)),
            pl.BlockSpec((BE, D_EDGE), lambda i: (i + blk0, 0)),
            pl.BlockSpec((BE, 1), lambda i: (i + blk0, 0)),
            pl.BlockSpec((BE, 1), lambda i: (i + blk0, 0)),
            full((D_EDGE, N_EXPERTS * HIDDEN)),
            full((N_EXPERTS, HIDDEN)),
            full((HIDDEN, N_EXPERTS * HIDDEN)),
            full((N_EXPERTS, HIDDEN)),
            full((HIDDEN, N_EXPERTS)),
            full((N_EXPERTS, 1)),
        ],
        out_specs=[
            pl.BlockSpec((BE, 1), lambda i: (i, 0)),
            pl.BlockSpec((BE, 1), lambda i: (i, 0)),
        ],
        out_shape=[
            jax.ShapeDtypeStruct((n_edges, 1), jnp.float32),
            jax.ShapeDtypeStruct((n_edges, 1), jnp.float32),
        ],
    )(hs, hd, edge_state, t3, bz3, w1c, b1, w2, b2, w3c, b3)


def kernel(node_embed, edge_state, edge_index, edge_type, baseline_z,
           W1, b1, W2, b2, W3, b3):
    pad = (0, E_PAD - N_EDGES)
    src = jnp.pad(edge_index[0].astype(jnp.int32), pad)
    dst = jnp.pad(edge_index[1].astype(jnp.int32), pad)
    etype = edge_type.astype(jnp.int32)
    etype_p = jnp.pad(etype, pad)

    w_stacked = jnp.concatenate([W1[:, :D_NODE, :], W1[:, D_NODE:2 * D_NODE, :]],
                                axis=0)
    table = _project_nodes(node_embed.astype(jnp.bfloat16),
                           w_stacked.astype(jnp.bfloat16)).reshape(
        2 * N_EXPERTS * N_NODES, HALF)

    w1c = W1[:, 2 * D_NODE:, :].transpose(1, 0, 2).reshape(
        D_EDGE, N_EXPERTS * HIDDEN).astype(jnp.bfloat16)
    w2c = W2.transpose(1, 0, 2).reshape(
        HIDDEN, N_EXPERTS * HIDDEN).astype(jnp.bfloat16)
    w3c = W3[:, :, 0].T
    t3 = etype.astype(jnp.float32).reshape(N_EDGES, 1)
    bz3 = baseline_z.reshape(N_EDGES, 1)

    dz_parts, rho_parts = [], []
    for s in range(NSLAB):
        lo = s * SLAB
        n = min(SLAB, N_EDGES - lo)
        hs, hd = _sc_gather(table, src, dst, etype_p, lo)
        dz_s, rho_s = _edge_mlp(n, lo // BE, hs, hd,
                                edge_state.astype(jnp.bfloat16), t3, bz3,
                                w1c, b1, w2c, b2, w3c, b3)
        dz_parts.append(dz_s)
        rho_parts.append(rho_s)

    dz3 = jnp.concatenate(dz_parts, axis=0)
    rho3 = jnp.concatenate(rho_parts, axis=0)
    return dz3.reshape(N_EDGES), rho3.reshape(N_EDGES)
